# baseline (speedup 1.0000x reference)
import functools

import jax
import jax.numpy as jnp
from jax import lax
from jax.experimental import pallas as pl
from jax.experimental.pallas import tpu as pltpu
from jax.experimental.pallas import tpu_sc as plsc

N = 10000
E = 320000
D_IN = 128
D_H = 256
K = 50
KP = 64

NC, NS = 2, 16
NW = NC * NS
EB = 128
BLKS = 80
E_PAD = NW * EB * BLKS
NROWS = E_PAD // EB
NPAD = 10112
RPT = NPAD // NS
RB = 1000
DW = 128

DEG_B0 = 80
AGG_B0 = 96
LOSS_B0 = 96


def _my_blocks(core, sub, b0):
    b1 = (NROWS - NS * b0) // NS
    start = jnp.where(core == 0, sub * b0, NS * b0 + sub * b1)
    cnt = jnp.where(core == 0, b0, b1)
    return start, cnt

_mesh = plsc.VectorSubcoreMesh(core_axis_name="c", subcore_axis_name="s")


def _wid():
    return lax.axis_index("c") * NS + lax.axis_index("s")


def _row_chunks():
    off = 0
    for sz in (128, 128, 128, 128, RPT - 4 * 128):
        yield off, sz
        off += sz


@functools.partial(
    pl.kernel,
    out_type=jax.ShapeDtypeStruct((NC * NPAD, DW), jnp.float32),
    mesh=_mesh,
    scratch_types=[
        pltpu.VMEM((DEG_B0, EB), jnp.int32),
        pltpu.VMEM((EB, DW), jnp.float32),
        pltpu.VMEM_SHARED((NPAD, DW), jnp.float32),
        pltpu.SemaphoreType.DMA,
    ],
)
def _sc_deg(dst_hbm, ones_hbm, zeros_hbm, out_hbm, idx_v, stage_v, acc_sh, sem):
    core = lax.axis_index("c")
    sub = lax.axis_index("s")
    row0 = sub * RPT
    start, cnt = _my_blocks(core, sub, DEG_B0)
    pltpu.sync_copy(zeros_hbm, stage_v)
    for off, sz in _row_chunks():
        pltpu.sync_copy(stage_v.at[pl.ds(0, sz)], acc_sh.at[pl.ds(row0 + off, sz)])
    plsc.subcore_barrier()
    pltpu.sync_copy(ones_hbm, stage_v)

    @pl.when(core == 0)
    def _():
        pltpu.sync_copy(dst_hbm.at[pl.ds(start, DEG_B0)], idx_v)

    @pl.when(core == 1)
    def _():
        _b1 = (NROWS - NS * DEG_B0) // NS
        pltpu.sync_copy(dst_hbm.at[pl.ds(start, _b1)], idx_v.at[pl.ds(0, _b1)])

    DG = 8

    def body(g, c):
        base = g * DG
        for i in range(DG):
            pltpu.async_copy(stage_v, acc_sh.at[idx_v.at[base + i]], sem,
                             add=True)

        @pl.when(g > 0)
        def _():
            for i in range(DG):
                pltpu.make_async_copy(
                    stage_v, acc_sh.at[idx_v.at[base - DG + i]], sem).wait()
        return c

    lax.fori_loop(0, cnt // DG, body, 0)
    for i in range(DG):
        pltpu.make_async_copy(
            stage_v, acc_sh.at[idx_v.at[cnt - DG + i]], sem).wait()
    plsc.subcore_barrier()
    for off, sz in _row_chunks():
        pltpu.sync_copy(acc_sh.at[pl.ds(row0 + off, sz)], stage_v.at[pl.ds(0, sz)])
        pltpu.sync_copy(stage_v.at[pl.ds(0, sz)],
                        out_hbm.at[pl.ds(core * NPAD + row0 + off, sz)])


SB = 8
NSB = BLKS // SB


def _make_sc_agg(D):
    @functools.partial(
        pl.kernel,
        out_type=jax.ShapeDtypeStruct((NC * NPAD, D), jnp.float32),
        mesh=_mesh,
        scratch_types=[
            pltpu.VMEM((SB, EB), jnp.int32),
            pltpu.VMEM((SB, EB), jnp.int32),
            pltpu.VMEM((EB, D), jnp.float32),
            pltpu.VMEM((EB, D), jnp.float32),
            pltpu.VMEM_SHARED((NPAD, D), jnp.float32),
            pltpu.SemaphoreType.DMA,
            pltpu.SemaphoreType.DMA,
        ],
    )
    def agg(src_hbm, dst_hbm, tab_hbm, zeros_hbm, out_hbm,
            sidx, didx, r0, r1, acc_sh, gsem, ssem):
        rows = (r0, r1)
        core = lax.axis_index("c")
        sub = lax.axis_index("s")
        row0 = sub * RPT
        start, cnt = _my_blocks(core, sub, AGG_B0)
        pltpu.sync_copy(zeros_hbm, r0)
        for off, sz in _row_chunks():
            pltpu.sync_copy(r0.at[pl.ds(0, sz)], acc_sh.at[pl.ds(row0 + off, sz)])
        plsc.subcore_barrier()

        def body(sb, c):
            base = start + sb * SB

            @pl.when(sb > 0)
            def _():
                for i in range(2):
                    pltpu.make_async_copy(
                        rows[i], acc_sh.at[didx.at[SB - 2 + i]], ssem).wait()
            pltpu.sync_copy(src_hbm.at[pl.ds(base, SB)], sidx)
            pltpu.sync_copy(dst_hbm.at[pl.ds(base, SB)], didx)
            for grp in range(SB // 2):
                if grp > 0:
                    for i in range(2):
                        pltpu.make_async_copy(
                            rows[i], acc_sh.at[didx.at[2 * grp - 2 + i]],
                            ssem).wait()
                for i in range(2):
                    pltpu.async_copy(tab_hbm.at[sidx.at[2 * grp + i]],
                                     rows[i], gsem)
                for i in range(2):
                    pltpu.make_async_copy(tab_hbm.at[sidx.at[2 * grp + i]],
                                          rows[i], gsem).wait()
                for i in range(2):
                    pltpu.async_copy(rows[i],
                                     acc_sh.at[didx.at[2 * grp + i]],
                                     ssem, add=True)
            return c

        lax.fori_loop(0, cnt // SB, body, 0)
        for i in range(2):
            pltpu.make_async_copy(
                rows[i], acc_sh.at[didx.at[SB - 2 + i]], ssem).wait()
        plsc.subcore_barrier()
        for off, sz in _row_chunks():
            pltpu.sync_copy(acc_sh.at[pl.ds(row0 + off, sz)], r0.at[pl.ds(0, sz)])
            pltpu.sync_copy(r0.at[pl.ds(0, sz)],
                            out_hbm.at[pl.ds(core * NPAD + row0 + off, sz)])

    return agg


_sc_agg128 = _make_sc_agg(D_IN)


@functools.partial(
    pl.kernel,
    out_type=jax.ShapeDtypeStruct((NW, 1, 16), jnp.float32),
    mesh=_mesh,
    scratch_types=[
        pltpu.VMEM((LOSS_B0, EB), jnp.int32),
        pltpu.VMEM((LOSS_B0, EB), jnp.int32),
        pltpu.VMEM((EB, D_IN), jnp.float32),
        pltpu.VMEM((EB, D_IN), jnp.float32),
        pltpu.VMEM((EB, D_IN), jnp.float32),
        pltpu.VMEM((EB, D_IN), jnp.float32),
        pltpu.VMEM((1, 16), jnp.float32),
        pltpu.SemaphoreType.DMA,
        pltpu.SemaphoreType.DMA,
    ],
)
def _sc_loss(src_hbm, dst_hbm, tab_hbm, out_hbm,
             sidx, didx, a0b, b0b, a1b, b1b, accv, sema, semb):
    core = lax.axis_index("c")
    sub = lax.axis_index("s")
    wid = core * NS + sub
    start, cnt = _my_blocks(core, sub, LOSS_B0)

    @pl.when(core == 0)
    def _():
        pltpu.sync_copy(src_hbm.at[pl.ds(start, LOSS_B0)], sidx)
        pltpu.sync_copy(dst_hbm.at[pl.ds(start, LOSS_B0)], didx)

    @pl.when(core == 1)
    def _():
        _b1 = (NROWS - NS * LOSS_B0) // NS
        pltpu.sync_copy(src_hbm.at[pl.ds(start, _b1)], sidx.at[pl.ds(0, _b1)])
        pltpu.sync_copy(dst_hbm.at[pl.ds(start, _b1)], didx.at[pl.ds(0, _b1)])
    msk = jnp.where(lax.iota(jnp.int32, 16) < 2,
                    jnp.full((16,), 1.0, jnp.float32),
                    jnp.zeros((16,), jnp.float32))

    def fire(j, ba, bb, sem):
        pltpu.async_copy(tab_hbm.at[sidx.at[j]], ba, sem)
        pltpu.async_copy(tab_hbm.at[didx.at[j]], bb, sem)

    def drain(j, ba, bb, sem):
        pltpu.make_async_copy(tab_hbm.at[sidx.at[j]], ba, sem).wait()
        pltpu.make_async_copy(tab_hbm.at[didx.at[j]], bb, sem).wait()

    def compute(bufa, bufb, acc):
        a0, a1, a2, a3 = acc
        for e in range(EB):
            va3 = bufa[e, 48:64]
            vb3 = bufb[e, 48:64]
            dv = va3 - vb3
            d2 = dv[2] * dv[2] + dv[3] * dv[3]
            a0 = a0 + (bufa[e, 0:16] * bufb[e, 0:16]) * d2
            a1 = a1 + (bufa[e, 16:32] * bufb[e, 16:32]) * d2
            a2 = a2 + (bufa[e, 32:48] * bufb[e, 32:48]) * d2
            a3 = a3 + (va3 * vb3 * msk) * d2
        return (a0, a1, a2, a3)

    nit = cnt // 2
    fire(0, a0b, b0b, sema)

    def body(i, acc):
        j0 = 2 * i
        fire(j0 + 1, a1b, b1b, semb)
        drain(j0, a0b, b0b, sema)
        acc = compute(a0b, b0b, acc)

        @pl.when(i < nit - 1)
        def _():
            fire(j0 + 2, a0b, b0b, sema)
        drain(j0 + 1, a1b, b1b, semb)
        acc = compute(a1b, b1b, acc)
        return acc

    z = jnp.zeros((16,), jnp.float32)
    a0, a1, a2, a3 = lax.fori_loop(0, nit, body, (z, z, z, z))
    accv[0, :] = a0 + a1 + a2 + a3
    pltpu.sync_copy(accv, out_hbm.at[wid])


def _tc1_body(d0, d1, x, xp, dinv8):
    deg = d0[...] + d1[...] + 1.0
    dinv = lax.rsqrt(deg)
    dinv8[...] = dinv[:, 0:8]
    xp[...] = x[...] * dinv[:, 0:1]


def _tc1(deg0, deg1, x):
    return pl.pallas_call(
        _tc1_body,
        grid=(N // RB,),
        in_specs=[
            pl.BlockSpec((RB, DW), lambda i: (i, 0)),
            pl.BlockSpec((RB, DW), lambda i: (i, 0)),
            pl.BlockSpec((RB, D_IN), lambda i: (i, 0)),
        ],
        out_specs=[
            pl.BlockSpec((RB, D_IN), lambda i: (i, 0)),
            pl.BlockSpec((RB, 8), lambda i: (i, 0)),
        ],
        out_shape=[
            jax.ShapeDtypeStruct((N, D_IN), jnp.float32),
            jax.ShapeDtypeStruct((N, 8), jnp.float32),
        ],
    )(deg0, deg1, x)


def _tc2_body(a0, a1, xp, dinv8, w1, b1, w2, zp):
    dinv = dinv8[:, 0:1]
    p = (a0[...] + a1[...] + xp[...]) * dinv
    h1 = jnp.maximum(
        jnp.dot(p, w1[...], preferred_element_type=jnp.float32) + b1[...], 0.0)
    z = jnp.dot(h1, w2[...], preferred_element_type=jnp.float32)
    zp[...] = z * dinv


def _tc2(a0, a1, xp, dinv8, w1, b1, w2p):
    return pl.pallas_call(
        _tc2_body,
        grid=(N // RB,),
        in_specs=[
            pl.BlockSpec((RB, D_IN), lambda i: (i, 0)),
            pl.BlockSpec((RB, D_IN), lambda i: (i, 0)),
            pl.BlockSpec((RB, D_IN), lambda i: (i, 0)),
            pl.BlockSpec((RB, 8), lambda i: (i, 0)),
            pl.BlockSpec((D_IN, D_H), lambda i: (0, 0)),
            pl.BlockSpec((1, D_H), lambda i: (0, 0)),
            pl.BlockSpec((D_H, KP), lambda i: (0, 0)),
        ],
        out_specs=pl.BlockSpec((RB, KP), lambda i: (i, 0)),
        out_shape=jax.ShapeDtypeStruct((N, KP), jnp.float32),
    )(a0, a1, xp, dinv8, w1, b1, w2p)


def _tc3_body(c0, c1, zp, dinv8, b2, pos, r):
    dinv = dinv8[:, 0:1]
    s = (c0[...] + c1[...] + zp[...]) * dinv + b2[...]
    col = lax.broadcasted_iota(jnp.int32, (RB, KP), 1)
    s = jnp.where(col < K, s, -1e30)
    m = jnp.max(s, axis=1, keepdims=True)
    ex = jnp.exp(s - m)
    sm = ex / jnp.sum(ex, axis=1, keepdims=True)
    px = pos[:, 0:1]
    py = pos[:, 1:2]
    r[...] = jnp.where(col == K, px, jnp.where(col == K + 1, py, sm))


def _tc3(c0, c1, zp, dinv8, b2p, pospad):
    return pl.pallas_call(
        _tc3_body,
        grid=(N // RB,),
        in_specs=[
            pl.BlockSpec((RB, KP), lambda i: (i, 0)),
            pl.BlockSpec((RB, KP), lambda i: (i, 0)),
            pl.BlockSpec((RB, KP), lambda i: (i, 0)),
            pl.BlockSpec((RB, 8), lambda i: (i, 0)),
            pl.BlockSpec((1, KP), lambda i: (0, 0)),
            pl.BlockSpec((RB, 8), lambda i: (i, 0)),
        ],
        out_specs=pl.BlockSpec((RB, KP), lambda i: (i, 0)),
        out_shape=jax.ShapeDtypeStruct((N, KP), jnp.float32),
    )(c0, c1, zp, dinv8, b2p, pospad)


def _tc4_body(part, sw, out):
    out[...] = jnp.sum(part[...]) * (1.0 / E) * sw[...]


def _tc4(part, sw):
    return pl.pallas_call(
        _tc4_body,
        out_shape=jax.ShapeDtypeStruct((1, 1), jnp.float32),
    )(part, sw)


def kernel(x, edge_index, positions, W1, b1, W2, b2, spatial_weight):
    f32 = jnp.float32
    src = edge_index[0]
    dst = edge_index[1]
    sentinel = jnp.full((E_PAD - E,), N, jnp.int32)
    srcp = jnp.concatenate([src, sentinel]).reshape(NROWS, EB)
    dstp = jnp.concatenate([dst, sentinel]).reshape(NROWS, EB)

    ones8 = jnp.zeros((EB, DW), f32).at[:, 0].set(1.0)
    zeros8 = jnp.zeros((EB, DW), f32)
    zeros128 = jnp.zeros((EB, D_IN), f32)

    degp = _sc_deg(dstp, ones8, zeros8)
    deg0 = degp[:N]
    deg1 = degp[NPAD:NPAD + N]

    xp, dinv8 = _tc1(deg0, deg1, x)
    xpad = jnp.concatenate([xp, jnp.zeros((NPAD - N, D_IN), f32)])

    acc1 = _sc_agg128(srcp, dstp, xpad, zeros128)
    a0 = acc1[:N]
    a1 = acc1[NPAD:NPAD + N]

    b1r = jnp.reshape(b1, (1, D_H))
    w2p = jnp.pad(W2[:, :K], ((0, 0), (0, KP - K)))
    zp = _tc2(a0, a1, xp, dinv8, W1, b1r, w2p)
    zpad = jnp.pad(zp, ((0, NPAD - N), (0, D_IN - KP)))

    acc2 = _sc_agg128(srcp, dstp, zpad, zeros128)
    c0 = acc2[:N, :KP]
    c1 = acc2[NPAD:NPAD + N, :KP]

    b2p = jnp.reshape(jnp.pad(b2[:K], (0, KP - K)), (1, KP))
    pospad = jnp.pad(positions, ((0, 0), (0, 6)))
    r = _tc3(c0, c1, zp, dinv8, b2p, pospad)
    rpad = jnp.pad(r, ((0, NPAD - N), (0, D_IN - KP)))

    part = _sc_loss(srcp, dstp, rpad)
    lmat = _tc4(part, jnp.reshape(spatial_weight, (1, 1)))

    return (r[:, :K], lmat[0, 0])


# speedup vs baseline: 1.0361x; 1.0361x over previous
import functools

import jax
import jax.numpy as jnp
from jax import lax
from jax.experimental import pallas as pl
from jax.experimental.pallas import tpu as pltpu
from jax.experimental.pallas import tpu_sc as plsc

N = 10000
E = 320000
D_IN = 128
D_H = 256
K = 50
KP = 64

NC, NS = 2, 16
NW = NC * NS
EB = 128
BLKS = 80
E_PAD = NW * EB * BLKS
NROWS = E_PAD // EB
NPAD = 10112
RPT = NPAD // NS
RB = 1000
DW = 128

DEG_B0 = 80
AGG_B0 = 112
LOSS_B0 = 112


def _my_blocks(core, sub, b0):
    b1 = (NROWS - NS * b0) // NS
    start = jnp.where(core == 0, sub * b0, NS * b0 + sub * b1)
    cnt = jnp.where(core == 0, b0, b1)
    return start, cnt

_mesh = plsc.VectorSubcoreMesh(core_axis_name="c", subcore_axis_name="s")


def _wid():
    return lax.axis_index("c") * NS + lax.axis_index("s")


def _row_chunks():
    off = 0
    for sz in (128, 128, 128, 128, RPT - 4 * 128):
        yield off, sz
        off += sz


@functools.partial(
    pl.kernel,
    out_type=jax.ShapeDtypeStruct((NC * NPAD, DW), jnp.float32),
    mesh=_mesh,
    scratch_types=[
        pltpu.VMEM((DEG_B0, EB), jnp.int32),
        pltpu.VMEM((EB, DW), jnp.float32),
        pltpu.VMEM_SHARED((NPAD, DW), jnp.float32),
        pltpu.SemaphoreType.DMA,
    ],
)
def _sc_deg(dst_hbm, ones_hbm, zeros_hbm, out_hbm, idx_v, stage_v, acc_sh, sem):
    core = lax.axis_index("c")
    sub = lax.axis_index("s")
    row0 = sub * RPT
    start, cnt = _my_blocks(core, sub, DEG_B0)
    pltpu.sync_copy(zeros_hbm, stage_v)
    for off, sz in _row_chunks():
        pltpu.sync_copy(stage_v.at[pl.ds(0, sz)], acc_sh.at[pl.ds(row0 + off, sz)])
    plsc.subcore_barrier()
    pltpu.sync_copy(ones_hbm, stage_v)

    @pl.when(core == 0)
    def _():
        pltpu.sync_copy(dst_hbm.at[pl.ds(start, DEG_B0)], idx_v)

    @pl.when(core == 1)
    def _():
        _b1 = (NROWS - NS * DEG_B0) // NS
        pltpu.sync_copy(dst_hbm.at[pl.ds(start, _b1)], idx_v.at[pl.ds(0, _b1)])

    DG = 8

    def body(g, c):
        base = g * DG
        for i in range(DG):
            pltpu.async_copy(stage_v, acc_sh.at[idx_v.at[base + i]], sem,
                             add=True)

        @pl.when(g > 0)
        def _():
            for i in range(DG):
                pltpu.make_async_copy(
                    stage_v, acc_sh.at[idx_v.at[base - DG + i]], sem).wait()
        return c

    lax.fori_loop(0, cnt // DG, body, 0)
    for i in range(DG):
        pltpu.make_async_copy(
            stage_v, acc_sh.at[idx_v.at[cnt - DG + i]], sem).wait()
    plsc.subcore_barrier()
    for off, sz in _row_chunks():
        pltpu.sync_copy(acc_sh.at[pl.ds(row0 + off, sz)], stage_v.at[pl.ds(0, sz)])
        pltpu.sync_copy(stage_v.at[pl.ds(0, sz)],
                        out_hbm.at[pl.ds(core * NPAD + row0 + off, sz)])


SB = 8
NSB = BLKS // SB


def _make_sc_agg(D):
    @functools.partial(
        pl.kernel,
        out_type=jax.ShapeDtypeStruct((NC * NPAD, D), jnp.float32),
        mesh=_mesh,
        scratch_types=[
            pltpu.VMEM((SB, EB), jnp.int32),
            pltpu.VMEM((SB, EB), jnp.int32),
            pltpu.VMEM((EB, D), jnp.float32),
            pltpu.VMEM((EB, D), jnp.float32),
            pltpu.VMEM_SHARED((NPAD, D), jnp.float32),
            pltpu.SemaphoreType.DMA,
            pltpu.SemaphoreType.DMA,
        ],
    )
    def agg(src_hbm, dst_hbm, tab_hbm, zeros_hbm, out_hbm,
            sidx, didx, r0, r1, acc_sh, gsem, ssem):
        rows = (r0, r1)
        core = lax.axis_index("c")
        sub = lax.axis_index("s")
        row0 = sub * RPT
        start, cnt = _my_blocks(core, sub, AGG_B0)
        pltpu.sync_copy(zeros_hbm, r0)
        for off, sz in _row_chunks():
            pltpu.sync_copy(r0.at[pl.ds(0, sz)], acc_sh.at[pl.ds(row0 + off, sz)])
        plsc.subcore_barrier()

        def body(sb, c):
            base = start + sb * SB

            @pl.when(sb > 0)
            def _():
                for i in range(2):
                    pltpu.make_async_copy(
                        rows[i], acc_sh.at[didx.at[SB - 2 + i]], ssem).wait()
            pltpu.sync_copy(src_hbm.at[pl.ds(base, SB)], sidx)
            pltpu.sync_copy(dst_hbm.at[pl.ds(base, SB)], didx)
            for grp in range(SB // 2):
                if grp > 0:
                    for i in range(2):
                        pltpu.make_async_copy(
                            rows[i], acc_sh.at[didx.at[2 * grp - 2 + i]],
                            ssem).wait()
                for i in range(2):
                    pltpu.async_copy(tab_hbm.at[sidx.at[2 * grp + i]],
                                     rows[i], gsem)
                for i in range(2):
                    pltpu.make_async_copy(tab_hbm.at[sidx.at[2 * grp + i]],
                                          rows[i], gsem).wait()
                for i in range(2):
                    pltpu.async_copy(rows[i],
                                     acc_sh.at[didx.at[2 * grp + i]],
                                     ssem, add=True)
            return c

        lax.fori_loop(0, cnt // SB, body, 0)
        for i in range(2):
            pltpu.make_async_copy(
                rows[i], acc_sh.at[didx.at[SB - 2 + i]], ssem).wait()
        plsc.subcore_barrier()
        for off, sz in _row_chunks():
            pltpu.sync_copy(acc_sh.at[pl.ds(row0 + off, sz)], r0.at[pl.ds(0, sz)])
            pltpu.sync_copy(r0.at[pl.ds(0, sz)],
                            out_hbm.at[pl.ds(core * NPAD + row0 + off, sz)])

    return agg


_sc_agg128 = _make_sc_agg(D_IN)


@functools.partial(
    pl.kernel,
    out_type=jax.ShapeDtypeStruct((NW, 1, 16), jnp.float32),
    mesh=_mesh,
    scratch_types=[
        pltpu.VMEM((LOSS_B0, EB), jnp.int32),
        pltpu.VMEM((LOSS_B0, EB), jnp.int32),
        pltpu.VMEM((EB, D_IN), jnp.float32),
        pltpu.VMEM((EB, D_IN), jnp.float32),
        pltpu.VMEM((EB, D_IN), jnp.float32),
        pltpu.VMEM((EB, D_IN), jnp.float32),
        pltpu.VMEM((1, 16), jnp.float32),
        pltpu.SemaphoreType.DMA,
        pltpu.SemaphoreType.DMA,
    ],
)
def _sc_loss(src_hbm, dst_hbm, tab_hbm, out_hbm,
             sidx, didx, a0b, b0b, a1b, b1b, accv, sema, semb):
    core = lax.axis_index("c")
    sub = lax.axis_index("s")
    wid = core * NS + sub
    start, cnt = _my_blocks(core, sub, LOSS_B0)

    @pl.when(core == 0)
    def _():
        pltpu.sync_copy(src_hbm.at[pl.ds(start, LOSS_B0)], sidx)
        pltpu.sync_copy(dst_hbm.at[pl.ds(start, LOSS_B0)], didx)

    @pl.when(core == 1)
    def _():
        _b1 = (NROWS - NS * LOSS_B0) // NS
        pltpu.sync_copy(src_hbm.at[pl.ds(start, _b1)], sidx.at[pl.ds(0, _b1)])
        pltpu.sync_copy(dst_hbm.at[pl.ds(start, _b1)], didx.at[pl.ds(0, _b1)])
    msk = jnp.where(lax.iota(jnp.int32, 16) < 2,
                    jnp.full((16,), 1.0, jnp.float32),
                    jnp.zeros((16,), jnp.float32))

    def fire(j, ba, bb, sem):
        pltpu.async_copy(tab_hbm.at[sidx.at[j]], ba, sem)
        pltpu.async_copy(tab_hbm.at[didx.at[j]], bb, sem)

    def drain(j, ba, bb, sem):
        pltpu.make_async_copy(tab_hbm.at[sidx.at[j]], ba, sem).wait()
        pltpu.make_async_copy(tab_hbm.at[didx.at[j]], bb, sem).wait()

    def compute(bufa, bufb, acc):
        a0, a1, a2, a3 = acc
        for e in range(EB):
            va3 = bufa[e, 48:64]
            vb3 = bufb[e, 48:64]
            dv = va3 - vb3
            d2 = dv[2] * dv[2] + dv[3] * dv[3]
            a0 = a0 + (bufa[e, 0:16] * bufb[e, 0:16]) * d2
            a1 = a1 + (bufa[e, 16:32] * bufb[e, 16:32]) * d2
            a2 = a2 + (bufa[e, 32:48] * bufb[e, 32:48]) * d2
            a3 = a3 + (va3 * vb3 * msk) * d2
        return (a0, a1, a2, a3)

    nit = cnt // 2
    fire(0, a0b, b0b, sema)

    def body(i, acc):
        j0 = 2 * i
        fire(j0 + 1, a1b, b1b, semb)
        drain(j0, a0b, b0b, sema)
        acc = compute(a0b, b0b, acc)

        @pl.when(i < nit - 1)
        def _():
            fire(j0 + 2, a0b, b0b, sema)
        drain(j0 + 1, a1b, b1b, semb)
        acc = compute(a1b, b1b, acc)
        return acc

    z = jnp.zeros((16,), jnp.float32)
    a0, a1, a2, a3 = lax.fori_loop(0, nit, body, (z, z, z, z))
    accv[0, :] = a0 + a1 + a2 + a3
    pltpu.sync_copy(accv, out_hbm.at[wid])


def _tc1_body(d0, d1, x, xp, dinv8):
    deg = d0[...] + d1[...] + 1.0
    dinv = lax.rsqrt(deg)
    dinv8[...] = dinv[:, 0:8]
    xp[...] = x[...] * dinv[:, 0:1]


def _tc1(deg0, deg1, x):
    return pl.pallas_call(
        _tc1_body,
        grid=(N // RB,),
        in_specs=[
            pl.BlockSpec((RB, DW), lambda i: (i, 0)),
            pl.BlockSpec((RB, DW), lambda i: (i, 0)),
            pl.BlockSpec((RB, D_IN), lambda i: (i, 0)),
        ],
        out_specs=[
            pl.BlockSpec((RB, D_IN), lambda i: (i, 0)),
            pl.BlockSpec((RB, 8), lambda i: (i, 0)),
        ],
        out_shape=[
            jax.ShapeDtypeStruct((N, D_IN), jnp.float32),
            jax.ShapeDtypeStruct((N, 8), jnp.float32),
        ],
    )(deg0, deg1, x)


def _tc2_body(a0, a1, xp, dinv8, w1, b1, w2, zp):
    dinv = dinv8[:, 0:1]
    p = (a0[...] + a1[...] + xp[...]) * dinv
    h1 = jnp.maximum(
        jnp.dot(p, w1[...], preferred_element_type=jnp.float32) + b1[...], 0.0)
    z = jnp.dot(h1, w2[...], preferred_element_type=jnp.float32)
    zp[...] = z * dinv


def _tc2(a0, a1, xp, dinv8, w1, b1, w2p):
    return pl.pallas_call(
        _tc2_body,
        grid=(N // RB,),
        in_specs=[
            pl.BlockSpec((RB, D_IN), lambda i: (i, 0)),
            pl.BlockSpec((RB, D_IN), lambda i: (i, 0)),
            pl.BlockSpec((RB, D_IN), lambda i: (i, 0)),
            pl.BlockSpec((RB, 8), lambda i: (i, 0)),
            pl.BlockSpec((D_IN, D_H), lambda i: (0, 0)),
            pl.BlockSpec((1, D_H), lambda i: (0, 0)),
            pl.BlockSpec((D_H, KP), lambda i: (0, 0)),
        ],
        out_specs=pl.BlockSpec((RB, KP), lambda i: (i, 0)),
        out_shape=jax.ShapeDtypeStruct((N, KP), jnp.float32),
    )(a0, a1, xp, dinv8, w1, b1, w2p)


def _tc3_body(c0, c1, zp, dinv8, b2, pos, r):
    dinv = dinv8[:, 0:1]
    s = (c0[...] + c1[...] + zp[...]) * dinv + b2[...]
    col = lax.broadcasted_iota(jnp.int32, (RB, KP), 1)
    s = jnp.where(col < K, s, -1e30)
    m = jnp.max(s, axis=1, keepdims=True)
    ex = jnp.exp(s - m)
    sm = ex / jnp.sum(ex, axis=1, keepdims=True)
    px = pos[:, 0:1]
    py = pos[:, 1:2]
    r[...] = jnp.where(col == K, px, jnp.where(col == K + 1, py, sm))


def _tc3(c0, c1, zp, dinv8, b2p, pospad):
    return pl.pallas_call(
        _tc3_body,
        grid=(N // RB,),
        in_specs=[
            pl.BlockSpec((RB, KP), lambda i: (i, 0)),
            pl.BlockSpec((RB, KP), lambda i: (i, 0)),
            pl.BlockSpec((RB, KP), lambda i: (i, 0)),
            pl.BlockSpec((RB, 8), lambda i: (i, 0)),
            pl.BlockSpec((1, KP), lambda i: (0, 0)),
            pl.BlockSpec((RB, 8), lambda i: (i, 0)),
        ],
        out_specs=pl.BlockSpec((RB, KP), lambda i: (i, 0)),
        out_shape=jax.ShapeDtypeStruct((N, KP), jnp.float32),
    )(c0, c1, zp, dinv8, b2p, pospad)


def _tc4_body(part, sw, out):
    out[...] = jnp.sum(part[...]) * (1.0 / E) * sw[...]


def _tc4(part, sw):
    return pl.pallas_call(
        _tc4_body,
        out_shape=jax.ShapeDtypeStruct((1, 1), jnp.float32),
    )(part, sw)


def kernel(x, edge_index, positions, W1, b1, W2, b2, spatial_weight):
    f32 = jnp.float32
    src = edge_index[0]
    dst = edge_index[1]
    sentinel = jnp.full((E_PAD - E,), N, jnp.int32)
    srcp = jnp.concatenate([src, sentinel]).reshape(NROWS, EB)
    dstp = jnp.concatenate([dst, sentinel]).reshape(NROWS, EB)

    ones8 = jnp.zeros((EB, DW), f32).at[:, 0].set(1.0)
    zeros8 = jnp.zeros((EB, DW), f32)
    zeros128 = jnp.zeros((EB, D_IN), f32)

    degp = _sc_deg(dstp, ones8, zeros8)
    deg0 = degp[:N]
    deg1 = degp[NPAD:NPAD + N]

    xp, dinv8 = _tc1(deg0, deg1, x)
    xpad = jnp.concatenate([xp, jnp.zeros((NPAD - N, D_IN), f32)])

    acc1 = _sc_agg128(srcp, dstp, xpad, zeros128)
    a0 = acc1[:N]
    a1 = acc1[NPAD:NPAD + N]

    b1r = jnp.reshape(b1, (1, D_H))
    w2p = jnp.pad(W2[:, :K], ((0, 0), (0, KP - K)))
    zp = _tc2(a0, a1, xp, dinv8, W1, b1r, w2p)
    zpad = jnp.pad(zp, ((0, NPAD - N), (0, D_IN - KP)))

    acc2 = _sc_agg128(srcp, dstp, zpad, zeros128)
    c0 = acc2[:N, :KP]
    c1 = acc2[NPAD:NPAD + N, :KP]

    b2p = jnp.reshape(jnp.pad(b2[:K], (0, KP - K)), (1, KP))
    pospad = jnp.pad(positions, ((0, 0), (0, 6)))
    r = _tc3(c0, c1, zp, dinv8, b2p, pospad)
    rpad = jnp.pad(r, ((0, NPAD - N), (0, D_IN - KP)))

    part = _sc_loss(srcp, dstp, rpad)
    lmat = _tc4(part, jnp.reshape(spatial_weight, (1, 1)))

    return (r[:, :K], lmat[0, 0])


# speedup vs baseline: 1.0380x; 1.0019x over previous
import functools

import jax
import jax.numpy as jnp
from jax import lax
from jax.experimental import pallas as pl
from jax.experimental.pallas import tpu as pltpu
from jax.experimental.pallas import tpu_sc as plsc

N = 10000
E = 320000
D_IN = 128
D_H = 256
K = 50
KP = 64

NC, NS = 2, 16
NW = NC * NS
EB = 128
BLKS = 80
E_PAD = NW * EB * BLKS
NROWS = E_PAD // EB
NPAD = 10112
RPT = NPAD // NS
RB = 1000
DW = 128

DEG_B0 = 80
AGG_B0 = 112
LOSS_B0 = 120


def _my_blocks(core, sub, b0):
    b1 = (NROWS - NS * b0) // NS
    start = jnp.where(core == 0, sub * b0, NS * b0 + sub * b1)
    cnt = jnp.where(core == 0, b0, b1)
    return start, cnt

_mesh = plsc.VectorSubcoreMesh(core_axis_name="c", subcore_axis_name="s")


def _wid():
    return lax.axis_index("c") * NS + lax.axis_index("s")


def _row_chunks():
    off = 0
    for sz in (128, 128, 128, 128, RPT - 4 * 128):
        yield off, sz
        off += sz


@functools.partial(
    pl.kernel,
    out_type=jax.ShapeDtypeStruct((NC * NPAD, DW), jnp.float32),
    mesh=_mesh,
    scratch_types=[
        pltpu.VMEM((DEG_B0, EB), jnp.int32),
        pltpu.VMEM((EB, DW), jnp.float32),
        pltpu.VMEM_SHARED((NPAD, DW), jnp.float32),
        pltpu.SemaphoreType.DMA,
    ],
)
def _sc_deg(dst_hbm, ones_hbm, zeros_hbm, out_hbm, idx_v, stage_v, acc_sh, sem):
    core = lax.axis_index("c")
    sub = lax.axis_index("s")
    row0 = sub * RPT
    start, cnt = _my_blocks(core, sub, DEG_B0)
    pltpu.sync_copy(zeros_hbm, stage_v)
    for off, sz in _row_chunks():
        pltpu.sync_copy(stage_v.at[pl.ds(0, sz)], acc_sh.at[pl.ds(row0 + off, sz)])
    plsc.subcore_barrier()
    pltpu.sync_copy(ones_hbm, stage_v)

    @pl.when(core == 0)
    def _():
        pltpu.sync_copy(dst_hbm.at[pl.ds(start, DEG_B0)], idx_v)

    @pl.when(core == 1)
    def _():
        _b1 = (NROWS - NS * DEG_B0) // NS
        pltpu.sync_copy(dst_hbm.at[pl.ds(start, _b1)], idx_v.at[pl.ds(0, _b1)])

    DG = 8

    def body(g, c):
        base = g * DG
        for i in range(DG):
            pltpu.async_copy(stage_v, acc_sh.at[idx_v.at[base + i]], sem,
                             add=True)

        @pl.when(g > 0)
        def _():
            for i in range(DG):
                pltpu.make_async_copy(
                    stage_v, acc_sh.at[idx_v.at[base - DG + i]], sem).wait()
        return c

    lax.fori_loop(0, cnt // DG, body, 0)
    for i in range(DG):
        pltpu.make_async_copy(
            stage_v, acc_sh.at[idx_v.at[cnt - DG + i]], sem).wait()
    plsc.subcore_barrier()
    for off, sz in _row_chunks():
        pltpu.sync_copy(acc_sh.at[pl.ds(row0 + off, sz)], stage_v.at[pl.ds(0, sz)])
        pltpu.sync_copy(stage_v.at[pl.ds(0, sz)],
                        out_hbm.at[pl.ds(core * NPAD + row0 + off, sz)])


SB = 8
NSB = BLKS // SB


def _make_sc_agg(D):
    @functools.partial(
        pl.kernel,
        out_type=jax.ShapeDtypeStruct((NC * NPAD, D), jnp.float32),
        mesh=_mesh,
        scratch_types=[
            pltpu.VMEM((SB, EB), jnp.int32),
            pltpu.VMEM((SB, EB), jnp.int32),
            pltpu.VMEM((EB, D), jnp.float32),
            pltpu.VMEM((EB, D), jnp.float32),
            pltpu.VMEM_SHARED((NPAD, D), jnp.float32),
            pltpu.SemaphoreType.DMA,
            pltpu.SemaphoreType.DMA,
        ],
    )
    def agg(src_hbm, dst_hbm, tab_hbm, zeros_hbm, out_hbm,
            sidx, didx, r0, r1, acc_sh, gsem, ssem):
        rows = (r0, r1)
        core = lax.axis_index("c")
        sub = lax.axis_index("s")
        row0 = sub * RPT
        start, cnt = _my_blocks(core, sub, AGG_B0)
        pltpu.sync_copy(zeros_hbm, r0)
        for off, sz in _row_chunks():
            pltpu.sync_copy(r0.at[pl.ds(0, sz)], acc_sh.at[pl.ds(row0 + off, sz)])
        plsc.subcore_barrier()

        def body(sb, c):
            base = start + sb * SB

            @pl.when(sb > 0)
            def _():
                for i in range(2):
                    pltpu.make_async_copy(
                        rows[i], acc_sh.at[didx.at[SB - 2 + i]], ssem).wait()
            pltpu.sync_copy(src_hbm.at[pl.ds(base, SB)], sidx)
            pltpu.sync_copy(dst_hbm.at[pl.ds(base, SB)], didx)
            for grp in range(SB // 2):
                if grp > 0:
                    for i in range(2):
                        pltpu.make_async_copy(
                            rows[i], acc_sh.at[didx.at[2 * grp - 2 + i]],
                            ssem).wait()
                for i in range(2):
                    pltpu.async_copy(tab_hbm.at[sidx.at[2 * grp + i]],
                                     rows[i], gsem)
                for i in range(2):
                    pltpu.make_async_copy(tab_hbm.at[sidx.at[2 * grp + i]],
                                          rows[i], gsem).wait()
                for i in range(2):
                    pltpu.async_copy(rows[i],
                                     acc_sh.at[didx.at[2 * grp + i]],
                                     ssem, add=True)
            return c

        lax.fori_loop(0, cnt // SB, body, 0)
        for i in range(2):
            pltpu.make_async_copy(
                rows[i], acc_sh.at[didx.at[SB - 2 + i]], ssem).wait()
        plsc.subcore_barrier()
        for off, sz in _row_chunks():
            pltpu.sync_copy(acc_sh.at[pl.ds(row0 + off, sz)], r0.at[pl.ds(0, sz)])
            pltpu.sync_copy(r0.at[pl.ds(0, sz)],
                            out_hbm.at[pl.ds(core * NPAD + row0 + off, sz)])

    return agg


_sc_agg128 = _make_sc_agg(D_IN)


@functools.partial(
    pl.kernel,
    out_type=jax.ShapeDtypeStruct((NW, 1, 16), jnp.float32),
    mesh=_mesh,
    scratch_types=[
        pltpu.VMEM((LOSS_B0, EB), jnp.int32),
        pltpu.VMEM((LOSS_B0, EB), jnp.int32),
        pltpu.VMEM((EB, D_IN), jnp.float32),
        pltpu.VMEM((EB, D_IN), jnp.float32),
        pltpu.VMEM((EB, D_IN), jnp.float32),
        pltpu.VMEM((EB, D_IN), jnp.float32),
        pltpu.VMEM((1, 16), jnp.float32),
        pltpu.SemaphoreType.DMA,
        pltpu.SemaphoreType.DMA,
    ],
)
def _sc_loss(src_hbm, dst_hbm, tab_hbm, out_hbm,
             sidx, didx, a0b, b0b, a1b, b1b, accv, sema, semb):
    core = lax.axis_index("c")
    sub = lax.axis_index("s")
    wid = core * NS + sub
    start, cnt = _my_blocks(core, sub, LOSS_B0)

    @pl.when(core == 0)
    def _():
        pltpu.sync_copy(src_hbm.at[pl.ds(start, LOSS_B0)], sidx)
        pltpu.sync_copy(dst_hbm.at[pl.ds(start, LOSS_B0)], didx)

    @pl.when(core == 1)
    def _():
        _b1 = (NROWS - NS * LOSS_B0) // NS
        pltpu.sync_copy(src_hbm.at[pl.ds(start, _b1)], sidx.at[pl.ds(0, _b1)])
        pltpu.sync_copy(dst_hbm.at[pl.ds(start, _b1)], didx.at[pl.ds(0, _b1)])
    msk = jnp.where(lax.iota(jnp.int32, 16) < 2,
                    jnp.full((16,), 1.0, jnp.float32),
                    jnp.zeros((16,), jnp.float32))

    def fire(j, ba, bb, sem):
        pltpu.async_copy(tab_hbm.at[sidx.at[j]], ba, sem)
        pltpu.async_copy(tab_hbm.at[didx.at[j]], bb, sem)

    def drain(j, ba, bb, sem):
        pltpu.make_async_copy(tab_hbm.at[sidx.at[j]], ba, sem).wait()
        pltpu.make_async_copy(tab_hbm.at[didx.at[j]], bb, sem).wait()

    def compute(bufa, bufb, acc):
        a0, a1, a2, a3 = acc
        for e in range(EB):
            va3 = bufa[e, 48:64]
            vb3 = bufb[e, 48:64]
            dv = va3 - vb3
            d2 = dv[2] * dv[2] + dv[3] * dv[3]
            a0 = a0 + (bufa[e, 0:16] * bufb[e, 0:16]) * d2
            a1 = a1 + (bufa[e, 16:32] * bufb[e, 16:32]) * d2
            a2 = a2 + (bufa[e, 32:48] * bufb[e, 32:48]) * d2
            a3 = a3 + (va3 * vb3 * msk) * d2
        return (a0, a1, a2, a3)

    nit = cnt // 2
    fire(0, a0b, b0b, sema)

    def body(i, acc):
        j0 = 2 * i
        fire(j0 + 1, a1b, b1b, semb)
        drain(j0, a0b, b0b, sema)
        acc = compute(a0b, b0b, acc)

        @pl.when(i < nit - 1)
        def _():
            fire(j0 + 2, a0b, b0b, sema)
        drain(j0 + 1, a1b, b1b, semb)
        acc = compute(a1b, b1b, acc)
        return acc

    z = jnp.zeros((16,), jnp.float32)
    a0, a1, a2, a3 = lax.fori_loop(0, nit, body, (z, z, z, z))
    accv[0, :] = a0 + a1 + a2 + a3
    pltpu.sync_copy(accv, out_hbm.at[wid])


def _tc1_body(d0, d1, x, xp, dinv8):
    deg = d0[...] + d1[...] + 1.0
    dinv = lax.rsqrt(deg)
    dinv8[...] = dinv[:, 0:8]
    xp[...] = x[...] * dinv[:, 0:1]


def _tc1(deg0, deg1, x):
    return pl.pallas_call(
        _tc1_body,
        grid=(N // RB,),
        in_specs=[
            pl.BlockSpec((RB, DW), lambda i: (i, 0)),
            pl.BlockSpec((RB, DW), lambda i: (i, 0)),
            pl.BlockSpec((RB, D_IN), lambda i: (i, 0)),
        ],
        out_specs=[
            pl.BlockSpec((RB, D_IN), lambda i: (i, 0)),
            pl.BlockSpec((RB, 8), lambda i: (i, 0)),
        ],
        out_shape=[
            jax.ShapeDtypeStruct((N, D_IN), jnp.float32),
            jax.ShapeDtypeStruct((N, 8), jnp.float32),
        ],
    )(deg0, deg1, x)


def _tc2_body(a0, a1, xp, dinv8, w1, b1, w2, zp):
    dinv = dinv8[:, 0:1]
    p = (a0[...] + a1[...] + xp[...]) * dinv
    h1 = jnp.maximum(
        jnp.dot(p, w1[...], preferred_element_type=jnp.float32) + b1[...], 0.0)
    z = jnp.dot(h1, w2[...], preferred_element_type=jnp.float32)
    zp[...] = z * dinv


def _tc2(a0, a1, xp, dinv8, w1, b1, w2p):
    return pl.pallas_call(
        _tc2_body,
        grid=(N // RB,),
        in_specs=[
            pl.BlockSpec((RB, D_IN), lambda i: (i, 0)),
            pl.BlockSpec((RB, D_IN), lambda i: (i, 0)),
            pl.BlockSpec((RB, D_IN), lambda i: (i, 0)),
            pl.BlockSpec((RB, 8), lambda i: (i, 0)),
            pl.BlockSpec((D_IN, D_H), lambda i: (0, 0)),
            pl.BlockSpec((1, D_H), lambda i: (0, 0)),
            pl.BlockSpec((D_H, KP), lambda i: (0, 0)),
        ],
        out_specs=pl.BlockSpec((RB, KP), lambda i: (i, 0)),
        out_shape=jax.ShapeDtypeStruct((N, KP), jnp.float32),
    )(a0, a1, xp, dinv8, w1, b1, w2p)


def _tc3_body(c0, c1, zp, dinv8, b2, pos, r):
    dinv = dinv8[:, 0:1]
    s = (c0[...] + c1[...] + zp[...]) * dinv + b2[...]
    col = lax.broadcasted_iota(jnp.int32, (RB, KP), 1)
    s = jnp.where(col < K, s, -1e30)
    m = jnp.max(s, axis=1, keepdims=True)
    ex = jnp.exp(s - m)
    sm = ex / jnp.sum(ex, axis=1, keepdims=True)
    px = pos[:, 0:1]
    py = pos[:, 1:2]
    r[...] = jnp.where(col == K, px, jnp.where(col == K + 1, py, sm))


def _tc3(c0, c1, zp, dinv8, b2p, pospad):
    return pl.pallas_call(
        _tc3_body,
        grid=(N // RB,),
        in_specs=[
            pl.BlockSpec((RB, KP), lambda i: (i, 0)),
            pl.BlockSpec((RB, KP), lambda i: (i, 0)),
            pl.BlockSpec((RB, KP), lambda i: (i, 0)),
            pl.BlockSpec((RB, 8), lambda i: (i, 0)),
            pl.BlockSpec((1, KP), lambda i: (0, 0)),
            pl.BlockSpec((RB, 8), lambda i: (i, 0)),
        ],
        out_specs=pl.BlockSpec((RB, KP), lambda i: (i, 0)),
        out_shape=jax.ShapeDtypeStruct((N, KP), jnp.float32),
    )(c0, c1, zp, dinv8, b2p, pospad)


def _tc4_body(part, sw, out):
    out[...] = jnp.sum(part[...]) * (1.0 / E) * sw[...]


def _tc4(part, sw):
    return pl.pallas_call(
        _tc4_body,
        out_shape=jax.ShapeDtypeStruct((1, 1), jnp.float32),
    )(part, sw)


def kernel(x, edge_index, positions, W1, b1, W2, b2, spatial_weight):
    f32 = jnp.float32
    src = edge_index[0]
    dst = edge_index[1]
    sentinel = jnp.full((E_PAD - E,), N, jnp.int32)
    srcp = jnp.concatenate([src, sentinel]).reshape(NROWS, EB)
    dstp = jnp.concatenate([dst, sentinel]).reshape(NROWS, EB)

    ones8 = jnp.zeros((EB, DW), f32).at[:, 0].set(1.0)
    zeros8 = jnp.zeros((EB, DW), f32)
    zeros128 = jnp.zeros((EB, D_IN), f32)

    degp = _sc_deg(dstp, ones8, zeros8)
    deg0 = degp[:N]
    deg1 = degp[NPAD:NPAD + N]

    xp, dinv8 = _tc1(deg0, deg1, x)
    xpad = jnp.concatenate([xp, jnp.zeros((NPAD - N, D_IN), f32)])

    acc1 = _sc_agg128(srcp, dstp, xpad, zeros128)
    a0 = acc1[:N]
    a1 = acc1[NPAD:NPAD + N]

    b1r = jnp.reshape(b1, (1, D_H))
    w2p = jnp.pad(W2[:, :K], ((0, 0), (0, KP - K)))
    zp = _tc2(a0, a1, xp, dinv8, W1, b1r, w2p)
    zpad = jnp.pad(zp, ((0, NPAD - N), (0, D_IN - KP)))

    acc2 = _sc_agg128(srcp, dstp, zpad, zeros128)
    c0 = acc2[:N, :KP]
    c1 = acc2[NPAD:NPAD + N, :KP]

    b2p = jnp.reshape(jnp.pad(b2[:K], (0, KP - K)), (1, KP))
    pospad = jnp.pad(positions, ((0, 0), (0, 6)))
    r = _tc3(c0, c1, zp, dinv8, b2p, pospad)
    rpad = jnp.pad(r, ((0, NPAD - N), (0, D_IN - KP)))

    part = _sc_loss(srcp, dstp, rpad)
    lmat = _tc4(part, jnp.reshape(spatial_weight, (1, 1)))

    return (r[:, :K], lmat[0, 0])
